# baseline (speedup 1.0000x reference)
import dataclasses
import functools

import jax
import jax.numpy as jnp
from jax import lax
from jax.experimental import pallas as pl
from jax.experimental.pallas import tpu as pltpu
from jax.experimental.pallas import tpu_sc as plsc

N = 10000
E = 320000
D = 128
BN_EPS = 1e-5

NC = 2
NS = 16
NW = NC * NS
EPW = E // NW
K = 128
NFULL = EPW // K
TAIL = EPW - NFULL * K
S0 = 624
SLAST = N - (NS - 1) * S0

_mesh = plsc.VectorSubcoreMesh(core_axis_name="c", subcore_axis_name="s")

L = 16
NROW = 80

_cp_novec = pltpu.CompilerParams()
if "needs_layout_passes" in pltpu.CompilerParams.__dataclass_fields__:
    _cp_novec = dataclasses.replace(_cp_novec, needs_layout_passes=False)


def _striped(s, fn):
    @pl.when(s < NS - 1)
    def _():
        fn(s * S0, S0)

    @pl.when(s == NS - 1)
    def _():
        fn((NS - 1) * S0, SLAST)


@functools.partial(
    pl.kernel,
    mesh=_mesh,
    compiler_params=_cp_novec,
    out_type=jax.ShapeDtypeStruct((NC, NROW, 128), jnp.float32),
    scratch_types=[
        pltpu.VMEM((K,), jnp.int32),
        pltpu.VMEM((K,), jnp.int32),
        pltpu.VMEM((TAIL,), jnp.int32),
        pltpu.VMEM((NROW,), jnp.int32),
        pltpu.VMEM((NROW, 128), jnp.float32),
        pltpu.VMEM_SHARED((NROW, 128), jnp.float32),
        pltpu.SemaphoreType.DMA,
        pltpu.SemaphoreType.DMA,
    ],
)
def _deg_kernel(dst_hbm, iota_hbm, zhist_hbm, out_hbm,
                idx0, idx1, idxt, iota_v, hist_v, acc_sh, sem0, sem1):
    c = lax.axis_index("c")
    s = lax.axis_index("s")
    wid = s * NC + c
    base0 = wid * EPW

    def load_idx(g, buf, sem):
        pltpu.async_copy(dst_hbm.at[pl.ds(base0 + g * K, K)], buf, sem)

    def wait_idx(g, buf, sem):
        pltpu.make_async_copy(dst_hbm.at[pl.ds(base0 + g * K, K)], buf, sem).wait()

    load_idx(0, idx0, sem0)
    load_idx(1, idx1, sem1)
    pltpu.sync_copy(iota_hbm, iota_v)
    pltpu.sync_copy(zhist_hbm, hist_v)

    @pl.when(s == 0)
    def _():
        pltpu.sync_copy(zhist_hbm, acc_sh)

    ones = jnp.ones((L,), jnp.float32)

    def hist_chunk(buf, nvec):
        @pl.loop(0, nvec)
        def _(i):
            iv = buf[pl.ds(i * L, L)]
            plsc.addupdate_scatter(hist_v, [iv >> 7, iv & 127], ones)

    wait_idx(0, idx0, sem0)

    @pl.loop(0, NFULL // 2)
    def _(p):
        ga = 2 * p
        gb = 2 * p + 1
        hist_chunk(idx0, K // L)

        @pl.when(p < NFULL // 2 - 1)
        def _():
            load_idx(ga + 2, idx0, sem0)

        wait_idx(gb, idx1, sem1)
        hist_chunk(idx1, K // L)

        @pl.when(p < NFULL // 2 - 1)
        def _():
            load_idx(gb + 2, idx1, sem1)
            wait_idx(ga + 2, idx0, sem0)

    pltpu.sync_copy(dst_hbm.at[pl.ds(base0 + NFULL * K, TAIL)], idxt)
    hist_chunk(idxt, TAIL // L)

    plsc.subcore_barrier()
    pltpu.sync_copy(hist_v, acc_sh.at[iota_v], add=True)
    plsc.subcore_barrier()

    @pl.when(s == 0)
    def _():
        pltpu.sync_copy(acc_sh, out_hbm.at[c])


@functools.partial(
    pl.kernel,
    mesh=_mesh,
    out_type=jax.ShapeDtypeStruct((NC, N, D), jnp.float32),
    scratch_types=[
        pltpu.VMEM((K,), jnp.int32),
        pltpu.VMEM((K,), jnp.int32),
        pltpu.VMEM((K,), jnp.int32),
        pltpu.VMEM((K,), jnp.int32),
        pltpu.VMEM((TAIL,), jnp.int32),
        pltpu.VMEM((TAIL,), jnp.int32),
        pltpu.VMEM((K, D), jnp.float32),
        pltpu.VMEM((K, D), jnp.float32),
        pltpu.VMEM_SHARED((N, D), jnp.float32),
        pltpu.SemaphoreType.DMA,
        pltpu.SemaphoreType.DMA,
        pltpu.SemaphoreType.DMA,
        pltpu.SemaphoreType.DMA,
        pltpu.SemaphoreType.DMA,
        pltpu.SemaphoreType.DMA,
    ],
)
def _edge_kernel(scaled_hbm, src_hbm, dst_hbm, zrow_hbm, out_hbm,
                 src0, src1, dst0, dst1, srct, dstt, rows0, rows1, acc_sh,
                 gsem0, gsem1, ss0, ss1, sd0, sd1):
    c = lax.axis_index("c")
    s = lax.axis_index("s")
    wid = s * NC + c
    base0 = wid * EPW
    half = NFULL // 2

    def load(arr, g, buf, sem):
        pltpu.async_copy(arr.at[pl.ds(base0 + g * K, K)], buf, sem)

    def drain(arr, g, buf, sem):
        pltpu.make_async_copy(arr.at[pl.ds(base0 + g * K, K)], buf, sem).wait()

    def gather(sbuf, rbuf, gsem):
        pltpu.async_copy(scaled_hbm.at[sbuf], rbuf, gsem)

    def wait_gather(sbuf, rbuf, gsem):
        pltpu.make_async_copy(scaled_hbm.at[sbuf], rbuf, gsem).wait()

    load(src_hbm, 0, src0, ss0)
    load(dst_hbm, 0, dst0, sd0)
    load(src_hbm, 1, src1, ss1)
    load(dst_hbm, 1, dst1, sd1)

    _striped(s, lambda o, n: pltpu.sync_copy(zrow_hbm.at[pl.ds(0, n)],
                                             acc_sh.at[pl.ds(o, n)]))

    drain(src_hbm, 0, src0, ss0)
    drain(dst_hbm, 0, dst0, sd0)
    drain(src_hbm, 1, src1, ss1)
    drain(dst_hbm, 1, dst1, sd1)
    plsc.subcore_barrier()

    gather(src0, rows0, gsem0)

    @pl.loop(0, half)
    def _(p):
        ga = 2 * p
        gb = 2 * p + 1
        @pl.when(p > 0)
        def _():
            drain(src_hbm, gb, src1, ss1)

        gather(src1, rows1, gsem1)
        wait_gather(src0, rows0, gsem0)

        @pl.when(p < half - 1)
        def _():
            load(src_hbm, ga + 2, src0, ss0)

        @pl.when(p > 0)
        def _():
            drain(dst_hbm, ga, dst0, sd0)

        pltpu.sync_copy(rows0, acc_sh.at[dst0], add=True)

        @pl.when(p < half - 1)
        def _():
            load(dst_hbm, ga + 2, dst0, sd0)
            drain(src_hbm, ga + 2, src0, ss0)
            gather(src0, rows0, gsem0)

        wait_gather(src1, rows1, gsem1)

        @pl.when(p > 0)
        def _():
            drain(dst_hbm, gb, dst1, sd1)

        pltpu.sync_copy(rows1, acc_sh.at[dst1], add=True)

        @pl.when(p < half - 1)
        def _():
            load(src_hbm, gb + 2, src1, ss1)
            load(dst_hbm, gb + 2, dst1, sd1)

    pltpu.sync_copy(src_hbm.at[pl.ds(base0 + NFULL * K, TAIL)], srct)
    pltpu.sync_copy(dst_hbm.at[pl.ds(base0 + NFULL * K, TAIL)], dstt)
    pltpu.async_copy(scaled_hbm.at[srct], rows0.at[pl.ds(0, TAIL)], gsem0).wait()
    pltpu.sync_copy(rows0.at[pl.ds(0, TAIL)], acc_sh.at[dstt], add=True)
    plsc.subcore_barrier()
    _striped(s, lambda o, n: pltpu.sync_copy(acc_sh.at[pl.ds(o, n)],
                                             out_hbm.at[c, pl.ds(o, n)]))


_BN = 1000


def _mm_scale_body(x_ref, w_ref, d_ref, o_ref):
    deg = d_ref[0] + d_ref[1] + 1.0
    xw = jnp.dot(x_ref[...], w_ref[...], preferred_element_type=jnp.float32)
    o_ref[...] = xw * lax.rsqrt(deg)


def _mm_scale(x, W, degn):
    return pl.pallas_call(
        _mm_scale_body,
        grid=(N // _BN,),
        in_specs=[pl.BlockSpec((_BN, D), lambda i: (i, 0)),
                  pl.BlockSpec((D, D), lambda i: (0, 0)),
                  pl.BlockSpec((NC, _BN, 1), lambda i: (0, i, 0))],
        out_specs=pl.BlockSpec((_BN, D), lambda i: (i, 0)),
        out_shape=jax.ShapeDtypeStruct((N, D), jnp.float32),
    )(x, W, degn)


def _final_body(p_ref, sc_ref, d_ref, lx_ref, b_ref, g_ref, bt_ref, rm_ref,
                rv_ref, o_ref):
    deg = d_ref[0] + d_ref[1] + 1.0
    dis = lax.rsqrt(deg)
    acc = (p_ref[0] + p_ref[1] + sc_ref[...]) * dis
    bn_scale = g_ref[...] * lax.rsqrt(rv_ref[...] + BN_EPS)
    h = (acc + b_ref[...] - rm_ref[...]) * bn_scale + bt_ref[...]
    o_ref[...] = jnp.maximum(h, 0.0) + lx_ref[...]


def _final(parts, scaled, degp, last_x, b, gamma, beta, rm, rv):
    vec = pl.BlockSpec((1, D), lambda i: (0, 0))
    return pl.pallas_call(
        _final_body,
        grid=(N // _BN,),
        in_specs=[pl.BlockSpec((NC, _BN, D), lambda i: (0, i, 0)),
                  pl.BlockSpec((_BN, D), lambda i: (i, 0)),
                  pl.BlockSpec((NC, _BN, 1), lambda i: (0, i, 0)),
                  pl.BlockSpec((_BN, D), lambda i: (i, 0)),
                  vec, vec, vec, vec, vec],
        out_specs=pl.BlockSpec((_BN, D), lambda i: (i, 0)),
        out_shape=jax.ShapeDtypeStruct((N, D), jnp.float32),
    )(parts, scaled, degp, last_x, b, gamma, beta, rm, rv)


def kernel(x, last_x, edge_index, W, b, gamma, beta, running_mean, running_var):
    src = edge_index[0]
    dst = edge_index[1]
    iota = jnp.arange(NROW, dtype=jnp.int32)
    zhist = jnp.zeros((NROW, 128), jnp.float32)
    zrow = jnp.zeros((SLAST, D), jnp.float32)

    degp = _deg_kernel(dst, iota, zhist)
    degn = degp.reshape(NC, NROW * 128)[:, :N].reshape(NC, N, 1)
    scaled = _mm_scale(x, W, degn)
    parts = _edge_kernel(scaled, src, dst, zrow)
    return _final(parts, scaled, degn, last_x,
                  b.reshape(1, D), gamma.reshape(1, D), beta.reshape(1, D),
                  running_mean.reshape(1, D), running_var.reshape(1, D))


# speedup vs baseline: 1.0303x; 1.0303x over previous
import dataclasses
import functools

import jax
import jax.numpy as jnp
from jax import lax
from jax.experimental import pallas as pl
from jax.experimental.pallas import tpu as pltpu
from jax.experimental.pallas import tpu_sc as plsc

N = 10000
E = 320000
D = 128
BN_EPS = 1e-5

NC = 2
NS = 16
NW = NC * NS
EPW = E // NW
K = 128
NFULL = EPW // K
TAIL = EPW - NFULL * K
S0 = 624
SLAST = N - (NS - 1) * S0

_mesh = plsc.VectorSubcoreMesh(core_axis_name="c", subcore_axis_name="s")

L = 16
NROW = 80

_cp_novec = pltpu.CompilerParams()
if "needs_layout_passes" in pltpu.CompilerParams.__dataclass_fields__:
    _cp_novec = dataclasses.replace(_cp_novec, needs_layout_passes=False)


def _striped(s, fn):
    @pl.when(s < NS - 1)
    def _():
        fn(s * S0, S0)

    @pl.when(s == NS - 1)
    def _():
        fn((NS - 1) * S0, SLAST)


@functools.partial(
    pl.kernel,
    mesh=_mesh,
    compiler_params=_cp_novec,
    out_type=jax.ShapeDtypeStruct((NC, NROW, 128), jnp.float32),
    scratch_types=[
        pltpu.VMEM((K,), jnp.int32),
        pltpu.VMEM((K,), jnp.int32),
        pltpu.VMEM((TAIL,), jnp.int32),
        pltpu.VMEM((NROW,), jnp.int32),
        pltpu.VMEM((NROW, 128), jnp.float32),
        pltpu.VMEM_SHARED((NROW, 128), jnp.float32),
        pltpu.SemaphoreType.DMA,
        pltpu.SemaphoreType.DMA,
    ],
)
def _deg_kernel(dst_hbm, iota_hbm, zhist_hbm, out_hbm,
                idx0, idx1, idxt, iota_v, hist_v, acc_sh, sem0, sem1):
    c = lax.axis_index("c")
    s = lax.axis_index("s")
    wid = s * NC + c
    base0 = wid * EPW

    def load_idx(g, buf, sem):
        pltpu.async_copy(dst_hbm.at[pl.ds(base0 + g * K, K)], buf, sem)

    def wait_idx(g, buf, sem):
        pltpu.make_async_copy(dst_hbm.at[pl.ds(base0 + g * K, K)], buf, sem).wait()

    load_idx(0, idx0, sem0)
    load_idx(1, idx1, sem1)
    pltpu.sync_copy(iota_hbm, iota_v)
    pltpu.sync_copy(zhist_hbm, hist_v)

    @pl.when(s == 0)
    def _():
        pltpu.sync_copy(zhist_hbm, acc_sh)

    ones = jnp.ones((L,), jnp.float32)

    def hist_chunk(buf, nvec):
        @pl.loop(0, nvec)
        def _(i):
            iv = buf[pl.ds(i * L, L)]
            plsc.addupdate_scatter(hist_v, [iv >> 7, iv & 127], ones)

    wait_idx(0, idx0, sem0)

    @pl.loop(0, NFULL // 2)
    def _(p):
        ga = 2 * p
        gb = 2 * p + 1
        hist_chunk(idx0, K // L)

        @pl.when(p < NFULL // 2 - 1)
        def _():
            load_idx(ga + 2, idx0, sem0)

        wait_idx(gb, idx1, sem1)
        hist_chunk(idx1, K // L)

        @pl.when(p < NFULL // 2 - 1)
        def _():
            load_idx(gb + 2, idx1, sem1)
            wait_idx(ga + 2, idx0, sem0)

    pltpu.sync_copy(dst_hbm.at[pl.ds(base0 + NFULL * K, TAIL)], idxt)
    hist_chunk(idxt, TAIL // L)

    plsc.subcore_barrier()
    pltpu.sync_copy(hist_v, acc_sh.at[iota_v], add=True)
    plsc.subcore_barrier()

    @pl.when(s == 0)
    def _():
        pltpu.sync_copy(acc_sh, out_hbm.at[c])


@functools.partial(
    pl.kernel,
    mesh=_mesh,
    out_type=jax.ShapeDtypeStruct((NC, N, D), jnp.float32),
    scratch_types=[
        pltpu.VMEM((K,), jnp.int32),
        pltpu.VMEM((K,), jnp.int32),
        pltpu.VMEM((K,), jnp.int32),
        pltpu.VMEM((K,), jnp.int32),
        pltpu.VMEM((TAIL,), jnp.int32),
        pltpu.VMEM((TAIL,), jnp.int32),
        pltpu.VMEM((K, D), jnp.float32),
        pltpu.VMEM((K, D), jnp.float32),
        pltpu.VMEM_SHARED((N, D), jnp.float32),
        pltpu.SemaphoreType.DMA,
        pltpu.SemaphoreType.DMA,
        pltpu.SemaphoreType.DMA,
        pltpu.SemaphoreType.DMA,
        pltpu.SemaphoreType.DMA,
        pltpu.SemaphoreType.DMA,
    ],
)
def _edge_kernel(scaled_hbm, src_hbm, dst_hbm, zrow_hbm, out_hbm,
                 src0, src1, dst0, dst1, srct, dstt, rows0, rows1, acc_sh,
                 gsem0, gsem1, ss0, ss1, sd0, sd1):
    c = lax.axis_index("c")
    s = lax.axis_index("s")
    wid = s * NC + c
    base0 = wid * EPW
    half = NFULL // 2

    def load(arr, g, buf, sem):
        pltpu.async_copy(arr.at[pl.ds(base0 + g * K, K)], buf, sem)

    def drain(arr, g, buf, sem):
        pltpu.make_async_copy(arr.at[pl.ds(base0 + g * K, K)], buf, sem).wait()

    def gather(sbuf, rbuf, gsem):
        pltpu.async_copy(scaled_hbm.at[sbuf], rbuf, gsem)

    def wait_gather(sbuf, rbuf, gsem):
        pltpu.make_async_copy(scaled_hbm.at[sbuf], rbuf, gsem).wait()

    load(src_hbm, 0, src0, ss0)
    load(dst_hbm, 0, dst0, sd0)
    load(src_hbm, 1, src1, ss1)
    load(dst_hbm, 1, dst1, sd1)

    @pl.when(c == 0)
    def _():
        _striped(s, lambda o, n: pltpu.sync_copy(scaled_hbm.at[pl.ds(o, n)],
                                                 acc_sh.at[pl.ds(o, n)]))

    @pl.when(c != 0)
    def _():
        _striped(s, lambda o, n: pltpu.sync_copy(zrow_hbm.at[pl.ds(0, n)],
                                                 acc_sh.at[pl.ds(o, n)]))

    drain(src_hbm, 0, src0, ss0)
    drain(dst_hbm, 0, dst0, sd0)
    drain(src_hbm, 1, src1, ss1)
    drain(dst_hbm, 1, dst1, sd1)
    plsc.subcore_barrier()

    gather(src0, rows0, gsem0)

    @pl.loop(0, half)
    def _(p):
        ga = 2 * p
        gb = 2 * p + 1
        @pl.when(p > 0)
        def _():
            drain(src_hbm, gb, src1, ss1)

        gather(src1, rows1, gsem1)
        wait_gather(src0, rows0, gsem0)

        @pl.when(p < half - 1)
        def _():
            load(src_hbm, ga + 2, src0, ss0)

        @pl.when(p > 0)
        def _():
            drain(dst_hbm, ga, dst0, sd0)

        pltpu.sync_copy(rows0, acc_sh.at[dst0], add=True)

        @pl.when(p < half - 1)
        def _():
            load(dst_hbm, ga + 2, dst0, sd0)
            drain(src_hbm, ga + 2, src0, ss0)
            gather(src0, rows0, gsem0)

        wait_gather(src1, rows1, gsem1)

        @pl.when(p > 0)
        def _():
            drain(dst_hbm, gb, dst1, sd1)

        pltpu.sync_copy(rows1, acc_sh.at[dst1], add=True)

        @pl.when(p < half - 1)
        def _():
            load(src_hbm, gb + 2, src1, ss1)
            load(dst_hbm, gb + 2, dst1, sd1)

    pltpu.sync_copy(src_hbm.at[pl.ds(base0 + NFULL * K, TAIL)], srct)
    pltpu.sync_copy(dst_hbm.at[pl.ds(base0 + NFULL * K, TAIL)], dstt)
    pltpu.async_copy(scaled_hbm.at[srct], rows0.at[pl.ds(0, TAIL)], gsem0).wait()
    pltpu.sync_copy(rows0.at[pl.ds(0, TAIL)], acc_sh.at[dstt], add=True)
    plsc.subcore_barrier()
    _striped(s, lambda o, n: pltpu.sync_copy(acc_sh.at[pl.ds(o, n)],
                                             out_hbm.at[c, pl.ds(o, n)]))


_BN = 2000


def _mm_scale_body(x_ref, w_ref, d_ref, o_ref):
    deg = d_ref[0] + d_ref[1] + 1.0
    xw = jnp.dot(x_ref[...], w_ref[...], preferred_element_type=jnp.float32)
    o_ref[...] = xw * lax.rsqrt(deg)


def _mm_scale(x, W, degn):
    return pl.pallas_call(
        _mm_scale_body,
        grid=(N // _BN,),
        in_specs=[pl.BlockSpec((_BN, D), lambda i: (i, 0)),
                  pl.BlockSpec((D, D), lambda i: (0, 0)),
                  pl.BlockSpec((NC, _BN, 1), lambda i: (0, i, 0))],
        out_specs=pl.BlockSpec((_BN, D), lambda i: (i, 0)),
        out_shape=jax.ShapeDtypeStruct((N, D), jnp.float32),
    )(x, W, degn)


def _final_body(p_ref, d_ref, lx_ref, b_ref, g_ref, bt_ref, rm_ref, rv_ref, o_ref):
    deg = d_ref[0] + d_ref[1] + 1.0
    dis = lax.rsqrt(deg)
    acc = (p_ref[0] + p_ref[1]) * dis
    bn_scale = g_ref[...] * lax.rsqrt(rv_ref[...] + BN_EPS)
    h = (acc + b_ref[...] - rm_ref[...]) * bn_scale + bt_ref[...]
    o_ref[...] = jnp.maximum(h, 0.0) + lx_ref[...]


def _final(parts, degp, last_x, b, gamma, beta, rm, rv):
    vec = pl.BlockSpec((1, D), lambda i: (0, 0))
    return pl.pallas_call(
        _final_body,
        grid=(N // _BN,),
        in_specs=[pl.BlockSpec((NC, _BN, D), lambda i: (0, i, 0)),
                  pl.BlockSpec((NC, _BN, 1), lambda i: (0, i, 0)),
                  pl.BlockSpec((_BN, D), lambda i: (i, 0)),
                  vec, vec, vec, vec, vec],
        out_specs=pl.BlockSpec((_BN, D), lambda i: (i, 0)),
        out_shape=jax.ShapeDtypeStruct((N, D), jnp.float32),
    )(parts, degp, last_x, b, gamma, beta, rm, rv)


def kernel(x, last_x, edge_index, W, b, gamma, beta, running_mean, running_var):
    src = edge_index[0]
    dst = edge_index[1]
    iota = jnp.arange(NROW, dtype=jnp.int32)
    zhist = jnp.zeros((NROW, 128), jnp.float32)
    zrow = jnp.zeros((SLAST, D), jnp.float32)

    degp = _deg_kernel(dst, iota, zhist)
    degn = degp.reshape(NC, NROW * 128)[:, :N].reshape(NC, N, 1)
    scaled = _mm_scale(x, W, degn)
    parts = _edge_kernel(scaled, src, dst, zrow)
    return _final(parts, degn, last_x,
                  b.reshape(1, D), gamma.reshape(1, D), beta.reshape(1, D),
                  running_mean.reshape(1, D), running_var.reshape(1, D))


# speedup vs baseline: 1.0334x; 1.0030x over previous
import dataclasses
import functools

import jax
import jax.numpy as jnp
from jax import lax
from jax.experimental import pallas as pl
from jax.experimental.pallas import tpu as pltpu
from jax.experimental.pallas import tpu_sc as plsc

N = 10000
E = 320000
D = 128
BN_EPS = 1e-5

NC = 2
NS = 16
NW = NC * NS
EPW = E // NW
K = 128
NFULL = EPW // K
TAIL = EPW - NFULL * K
S0 = 624
SLAST = N - (NS - 1) * S0

_mesh = plsc.VectorSubcoreMesh(core_axis_name="c", subcore_axis_name="s")

L = 16
NROW = 80

_cp_novec = pltpu.CompilerParams()
if "needs_layout_passes" in pltpu.CompilerParams.__dataclass_fields__:
    _cp_novec = dataclasses.replace(_cp_novec, needs_layout_passes=False)


def _striped(s, fn):
    @pl.when(s < NS - 1)
    def _():
        fn(s * S0, S0)

    @pl.when(s == NS - 1)
    def _():
        fn((NS - 1) * S0, SLAST)


@functools.partial(
    pl.kernel,
    mesh=_mesh,
    compiler_params=_cp_novec,
    out_type=jax.ShapeDtypeStruct((NC, NROW, 128), jnp.float32),
    scratch_types=[
        pltpu.VMEM((K,), jnp.int32),
        pltpu.VMEM((K,), jnp.int32),
        pltpu.VMEM((TAIL,), jnp.int32),
        pltpu.VMEM((NROW,), jnp.int32),
        pltpu.VMEM((NROW, 128), jnp.float32),
        pltpu.VMEM_SHARED((NROW, 128), jnp.float32),
        pltpu.SemaphoreType.DMA,
        pltpu.SemaphoreType.DMA,
    ],
)
def _deg_kernel(dst_hbm, iota_hbm, zhist_hbm, out_hbm,
                idx0, idx1, idxt, iota_v, hist_v, acc_sh, sem0, sem1):
    c = lax.axis_index("c")
    s = lax.axis_index("s")
    wid = s * NC + c
    base0 = wid * EPW

    def load_idx(g, buf, sem):
        pltpu.async_copy(dst_hbm.at[pl.ds(base0 + g * K, K)], buf, sem)

    def wait_idx(g, buf, sem):
        pltpu.make_async_copy(dst_hbm.at[pl.ds(base0 + g * K, K)], buf, sem).wait()

    load_idx(0, idx0, sem0)
    load_idx(1, idx1, sem1)
    pltpu.sync_copy(iota_hbm, iota_v)
    pltpu.sync_copy(zhist_hbm, hist_v)

    @pl.when(s == 0)
    def _():
        pltpu.sync_copy(zhist_hbm, acc_sh)

    ones = jnp.ones((L,), jnp.float32)

    def hist_chunk(buf, nvec):
        @pl.loop(0, nvec)
        def _(i):
            iv = buf[pl.ds(i * L, L)]
            plsc.addupdate_scatter(hist_v, [iv >> 7, iv & 127], ones)

    wait_idx(0, idx0, sem0)

    @pl.loop(0, NFULL // 2)
    def _(p):
        ga = 2 * p
        gb = 2 * p + 1
        hist_chunk(idx0, K // L)

        @pl.when(p < NFULL // 2 - 1)
        def _():
            load_idx(ga + 2, idx0, sem0)

        wait_idx(gb, idx1, sem1)
        hist_chunk(idx1, K // L)

        @pl.when(p < NFULL // 2 - 1)
        def _():
            load_idx(gb + 2, idx1, sem1)
            wait_idx(ga + 2, idx0, sem0)

    pltpu.sync_copy(dst_hbm.at[pl.ds(base0 + NFULL * K, TAIL)], idxt)
    hist_chunk(idxt, TAIL // L)

    plsc.subcore_barrier()
    pltpu.sync_copy(hist_v, acc_sh.at[iota_v], add=True)
    plsc.subcore_barrier()

    @pl.when(s == 0)
    def _():
        pltpu.sync_copy(acc_sh, out_hbm.at[c])


@functools.partial(
    pl.kernel,
    mesh=_mesh,
    out_type=jax.ShapeDtypeStruct((NC, N, D), jnp.float32),
    scratch_types=[
        pltpu.VMEM((K,), jnp.int32),
        pltpu.VMEM((K,), jnp.int32),
        pltpu.VMEM((K,), jnp.int32),
        pltpu.VMEM((K,), jnp.int32),
        pltpu.VMEM((TAIL,), jnp.int32),
        pltpu.VMEM((TAIL,), jnp.int32),
        pltpu.VMEM((K, D), jnp.float32),
        pltpu.VMEM((K, D), jnp.float32),
        pltpu.VMEM_SHARED((N, D), jnp.float32),
        pltpu.SemaphoreType.DMA,
        pltpu.SemaphoreType.DMA,
        pltpu.SemaphoreType.DMA,
        pltpu.SemaphoreType.DMA,
        pltpu.SemaphoreType.DMA,
        pltpu.SemaphoreType.DMA,
    ],
)
def _edge_kernel(scaled_hbm, src_hbm, dst_hbm, zrow_hbm, out_hbm,
                 src0, src1, dst0, dst1, srct, dstt, rows0, rows1, acc_sh,
                 gsem0, gsem1, ss0, ss1, sd0, sd1):
    c = lax.axis_index("c")
    s = lax.axis_index("s")
    wid = s * NC + c
    base0 = wid * EPW
    half = NFULL // 2

    def load(arr, g, buf, sem):
        pltpu.async_copy(arr.at[pl.ds(base0 + g * K, K)], buf, sem)

    def drain(arr, g, buf, sem):
        pltpu.make_async_copy(arr.at[pl.ds(base0 + g * K, K)], buf, sem).wait()

    def gather(sbuf, rbuf, gsem):
        pltpu.async_copy(scaled_hbm.at[sbuf], rbuf, gsem)

    def wait_gather(sbuf, rbuf, gsem):
        pltpu.make_async_copy(scaled_hbm.at[sbuf], rbuf, gsem).wait()

    load(src_hbm, 0, src0, ss0)
    load(dst_hbm, 0, dst0, sd0)
    load(src_hbm, 1, src1, ss1)
    load(dst_hbm, 1, dst1, sd1)

    @pl.when(c == 0)
    def _():
        _striped(s, lambda o, n: pltpu.sync_copy(scaled_hbm.at[pl.ds(o, n)],
                                                 acc_sh.at[pl.ds(o, n)]))

    @pl.when(c != 0)
    def _():
        _striped(s, lambda o, n: pltpu.sync_copy(zrow_hbm.at[pl.ds(0, n)],
                                                 acc_sh.at[pl.ds(o, n)]))

    drain(src_hbm, 0, src0, ss0)
    drain(dst_hbm, 0, dst0, sd0)
    drain(src_hbm, 1, src1, ss1)
    drain(dst_hbm, 1, dst1, sd1)
    plsc.subcore_barrier()

    gather(src0, rows0, gsem0)

    @pl.loop(0, half)
    def _(p):
        ga = 2 * p
        gb = 2 * p + 1
        @pl.when(p > 0)
        def _():
            drain(src_hbm, gb, src1, ss1)

        gather(src1, rows1, gsem1)
        wait_gather(src0, rows0, gsem0)

        @pl.when(p < half - 1)
        def _():
            load(src_hbm, ga + 2, src0, ss0)

        @pl.when(p > 0)
        def _():
            drain(dst_hbm, ga, dst0, sd0)

        pltpu.sync_copy(rows0, acc_sh.at[dst0], add=True)

        @pl.when(p < half - 1)
        def _():
            load(dst_hbm, ga + 2, dst0, sd0)
            drain(src_hbm, ga + 2, src0, ss0)
            gather(src0, rows0, gsem0)

        wait_gather(src1, rows1, gsem1)

        @pl.when(p > 0)
        def _():
            drain(dst_hbm, gb, dst1, sd1)

        pltpu.sync_copy(rows1, acc_sh.at[dst1], add=True)

        @pl.when(p < half - 1)
        def _():
            load(src_hbm, gb + 2, src1, ss1)
            load(dst_hbm, gb + 2, dst1, sd1)

    pltpu.sync_copy(src_hbm.at[pl.ds(base0 + NFULL * K, TAIL)], srct)
    pltpu.sync_copy(dst_hbm.at[pl.ds(base0 + NFULL * K, TAIL)], dstt)
    pltpu.async_copy(scaled_hbm.at[srct], rows0.at[pl.ds(0, TAIL)], gsem0).wait()
    pltpu.sync_copy(rows0.at[pl.ds(0, TAIL)], acc_sh.at[dstt], add=True)
    plsc.subcore_barrier()
    _striped(s, lambda o, n: pltpu.sync_copy(acc_sh.at[pl.ds(o, n)],
                                             out_hbm.at[c, pl.ds(o, n)]))


_BN = 5000


def _mm_scale_body(x_ref, w_ref, d_ref, o_ref):
    deg = d_ref[0] + d_ref[1] + 1.0
    xw = jnp.dot(x_ref[...], w_ref[...], preferred_element_type=jnp.float32)
    o_ref[...] = xw * lax.rsqrt(deg)


def _mm_scale(x, W, degn):
    return pl.pallas_call(
        _mm_scale_body,
        grid=(N // _BN,),
        in_specs=[pl.BlockSpec((_BN, D), lambda i: (i, 0)),
                  pl.BlockSpec((D, D), lambda i: (0, 0)),
                  pl.BlockSpec((NC, _BN, 1), lambda i: (0, i, 0))],
        out_specs=pl.BlockSpec((_BN, D), lambda i: (i, 0)),
        out_shape=jax.ShapeDtypeStruct((N, D), jnp.float32),
    )(x, W, degn)


def _final_body(p_ref, d_ref, lx_ref, b_ref, g_ref, bt_ref, rm_ref, rv_ref, o_ref):
    deg = d_ref[0] + d_ref[1] + 1.0
    dis = lax.rsqrt(deg)
    acc = (p_ref[0] + p_ref[1]) * dis
    bn_scale = g_ref[...] * lax.rsqrt(rv_ref[...] + BN_EPS)
    h = (acc + b_ref[...] - rm_ref[...]) * bn_scale + bt_ref[...]
    o_ref[...] = jnp.maximum(h, 0.0) + lx_ref[...]


def _final(parts, degp, last_x, b, gamma, beta, rm, rv):
    vec = pl.BlockSpec((1, D), lambda i: (0, 0))
    return pl.pallas_call(
        _final_body,
        grid=(N // _BN,),
        in_specs=[pl.BlockSpec((NC, _BN, D), lambda i: (0, i, 0)),
                  pl.BlockSpec((NC, _BN, 1), lambda i: (0, i, 0)),
                  pl.BlockSpec((_BN, D), lambda i: (i, 0)),
                  vec, vec, vec, vec, vec],
        out_specs=pl.BlockSpec((_BN, D), lambda i: (i, 0)),
        out_shape=jax.ShapeDtypeStruct((N, D), jnp.float32),
    )(parts, degp, last_x, b, gamma, beta, rm, rv)


def kernel(x, last_x, edge_index, W, b, gamma, beta, running_mean, running_var):
    src = edge_index[0]
    dst = edge_index[1]
    iota = jnp.arange(NROW, dtype=jnp.int32)
    zhist = jnp.zeros((NROW, 128), jnp.float32)
    zrow = jnp.zeros((SLAST, D), jnp.float32)

    degp = _deg_kernel(dst, iota, zhist)
    degn = degp.reshape(NC, NROW * 128)[:, :N].reshape(NC, N, 1)
    scaled = _mm_scale(x, W, degn)
    parts = _edge_kernel(scaled, src, dst, zrow)
    return _final(parts, degn, last_x,
                  b.reshape(1, D), gamma.reshape(1, D), beta.reshape(1, D),
                  running_mean.reshape(1, D), running_var.reshape(1, D))
